# baseline (speedup 1.0000x reference)
import functools

import jax
import jax.numpy as jnp
from jax import lax
from jax.experimental import pallas as pl
from jax.experimental.pallas import tpu as pltpu
from jax.experimental.pallas import tpu_sc as plsc

N = 5850
E = 93600
NP = 5888
RPT = 368
NW = 32
CH = 128
NCH = 23
EW = NCH * CH
EP = NW * EW
D1 = 8
D2 = 24

_sc_params = pltpu.CompilerParams(use_tc_tiling_on_sc=False)


@functools.cache
def _mesh():
    return plsc.VectorSubcoreMesh(
        core_axis_name="c", subcore_axis_name="s", num_cores=2, num_subcores=16
    )


def _leaky(v):
    return jnp.where(v >= 0, v, 0.01 * v)


def _sc_degree(dstp):
    @functools.partial(
        pl.kernel,
        out_type=jax.ShapeDtypeStruct((2, NP, D1), jnp.float32),
        mesh=_mesh(),
        compiler_params=_sc_params,
        scratch_types=[
            pltpu.VMEM((NCH, CH), jnp.int32),
            pltpu.VMEM((CH, D1), jnp.float32),
            pltpu.VMEM_SHARED((NP, D1), jnp.float32),
            pltpu.SemaphoreType.DMA,
        ],
    )
    def k(dstp_hbm, z_hbm, ones_hbm, out_hbm, idx_d, onesb, acc, sem):
        c = lax.axis_index("c")
        s = lax.axis_index("s")
        wid = c * 16 + s
        row = pl.ds(s * RPT, RPT)
        pltpu.sync_copy(z_hbm, acc.at[row, :])
        pltpu.sync_copy(ones_hbm, onesb)
        pltpu.sync_copy(dstp_hbm.at[wid], idx_d)
        plsc.subcore_barrier()
        for j in range(NCH):
            pltpu.sync_copy(onesb, acc.at[idx_d.at[j]], add=True)
        plsc.subcore_barrier()
        pltpu.sync_copy(acc.at[row, :], out_hbm.at[c].at[row, :])

    return k(dstp, jnp.zeros((RPT, D1), jnp.float32),
             jnp.ones((CH, D1), jnp.float32))


def _sc_scatter(g, srcp, dstp, d):
    @functools.partial(
        pl.kernel,
        out_type=jax.ShapeDtypeStruct((2, NP, d), jnp.float32),
        mesh=_mesh(),
        compiler_params=_sc_params,
        scratch_types=[
            pltpu.VMEM((NCH, CH), jnp.int32),
            pltpu.VMEM((NCH, CH), jnp.int32),
            pltpu.VMEM((EW, d), jnp.float32),
            pltpu.VMEM_SHARED((NP, d), jnp.float32),
            pltpu.VMEM_SHARED((NP, d), jnp.float32),
            pltpu.SemaphoreType.DMA,
        ],
    )
    def k(g_hbm, srcp_hbm, dstp_hbm, z_hbm, out_hbm, idx_s, idx_d, rows, g_sh, acc, sem):
        c = lax.axis_index("c")
        s = lax.axis_index("s")
        wid = c * 16 + s
        row = pl.ds(s * RPT, RPT)
        pltpu.sync_copy(z_hbm, acc.at[row, :])
        pltpu.sync_copy(g_hbm.at[row, :], g_sh.at[row, :])
        pltpu.sync_copy(srcp_hbm.at[wid], idx_s)
        pltpu.sync_copy(dstp_hbm.at[wid], idx_d)
        plsc.subcore_barrier()
        cps = [
            pltpu.async_copy(g_sh.at[idx_s.at[j]], rows.at[pl.ds(j * CH, CH), :], sem)
            for j in range(NCH)
        ]
        for cp in cps:
            cp.wait()
        for j in range(NCH):
            pltpu.sync_copy(rows.at[pl.ds(j * CH, CH), :], acc.at[idx_d.at[j]], add=True)
        plsc.subcore_barrier()
        pltpu.sync_copy(acc.at[row, :], out_hbm.at[c].at[row, :])

    return k(g, srcp, dstp, jnp.zeros((RPT, d), jnp.float32))


def _tc_h1(posp, W1p):
    def body(pos_ref, w_ref, h1_ref):
        h1_ref[...] = jnp.dot(pos_ref[...], w_ref[...],
                              preferred_element_type=jnp.float32)

    return pl.pallas_call(
        body,
        out_shape=jax.ShapeDtypeStruct((NP, D1), jnp.float32),
    )(posp, W1p)


def _tc_scale(h1, degp):
    def body(h1_ref, deg_ref, g1_ref, dinv_ref):
        deg = deg_ref[0, :, 0:1] + deg_ref[1, :, 0:1] + 1.0
        dinv = lax.rsqrt(deg)
        g1_ref[...] = dinv * h1_ref[...]
        dinv_ref[...] = dinv

    return pl.pallas_call(
        body,
        out_shape=(
            jax.ShapeDtypeStruct((NP, D1), jnp.float32),
            jax.ShapeDtypeStruct((NP, 1), jnp.float32),
        ),
    )(h1, degp)


def _tc_mid(s1p, g1, dinv, b1p, W2p):
    def body(s_ref, g1_ref, dinv_ref, b1_ref, w2_ref, g2_ref):
        dinv = dinv_ref[...]
        x1 = _leaky(dinv * (s_ref[0] + s_ref[1] + g1_ref[...]) + b1_ref[...])
        h2 = jnp.dot(x1, w2_ref[...], preferred_element_type=jnp.float32)
        g2_ref[...] = dinv * h2

    return pl.pallas_call(
        body,
        out_shape=jax.ShapeDtypeStruct((NP, D2), jnp.float32),
    )(s1p, g1, dinv, b1p, W2p)


_NB = 585
_KSTEPS = 10


def _tc_post_fc2(s2n, g2n, dinvn, b2p, fc1_Wp, fc1_b, W4, fc2_b2d):
    def body(s_ref, g2_ref, dinv_ref, b2_ref, w1_ref, b1_ref, w4_ref, b4_ref, out_ref):
        kk = pl.program_id(0)
        x2 = _leaky(dinv_ref[0] * (s_ref[0, 0] + s_ref[1, 0] + g2_ref[0])
                    + b2_ref[...])
        x3 = _leaky(jnp.dot(x2, w1_ref[...], preferred_element_type=jnp.float32)
                    + b1_ref[...])
        x3t = x3.T
        part = sum(
            jnp.dot(x3t[j:j + 1, :], w4_ref[0, :, j, :],
                    preferred_element_type=jnp.float32)
            for j in range(10)
        )
        acc = jnp.where(kk == 0, part, out_ref[...] + part)
        out_ref[...] = jnp.where(kk == _KSTEPS - 1, _leaky(acc + b4_ref[...]), acc)

    return pl.pallas_call(
        body,
        grid=(_KSTEPS,),
        in_specs=[
            pl.BlockSpec((2, 1, _NB, D2), lambda k: (0, k, 0, 0)),
            pl.BlockSpec((1, _NB, D2), lambda k: (k, 0, 0)),
            pl.BlockSpec((1, _NB, 1), lambda k: (k, 0, 0)),
            pl.BlockSpec((D2,), lambda k: (0,)),
            pl.BlockSpec((D2, 10), lambda k: (0, 0)),
            pl.BlockSpec((10,), lambda k: (0,)),
            pl.BlockSpec((1, _NB, 10, 100), lambda k: (k, 0, 0, 0)),
            pl.BlockSpec((1, 100), lambda k: (0, 0)),
        ],
        out_specs=pl.BlockSpec((1, 100), lambda k: (0, 0)),
        out_shape=jax.ShapeDtypeStruct((1, 100), jnp.float32),
    )(s2n, g2n, dinvn, b2p, fc1_Wp, fc1_b, W4, fc2_b2d)


def kernel(pos, edge_index, W1, b1, W2, b2, fc1_W, fc1_b, fc2_W, fc2_b):
    pad = N + (jnp.arange(EP - E, dtype=jnp.int32) % 8)
    srcp = jnp.concatenate([edge_index[0], pad]).reshape(NW, NCH, CH)
    dstp = jnp.concatenate([edge_index[1], pad]).reshape(NW, NCH, CH)
    posp = jnp.pad(pos, ((0, NP - N), (0, 0)))
    W1p = jnp.pad(W1, ((0, 0), (0, D1 - 5)))
    b1p = jnp.pad(b1, (0, D1 - 5))
    W2p = jnp.pad(W2, ((0, D1 - 5), (0, D2 - 20)))
    b2p = jnp.pad(b2, (0, D2 - 20))
    fc1_Wp = jnp.pad(fc1_W, ((0, D2 - 20), (0, 0)))

    h1 = _tc_h1(posp, W1p)
    degp = _sc_degree(dstp)
    g1, dinv = _tc_scale(h1, degp)
    s1p = _sc_scatter(g1, srcp, dstp, D1)
    g2 = _tc_mid(s1p, g1, dinv, b1p, W2p)
    s2p = _sc_scatter(g2, srcp, dstp, D2)
    s2n = s2p[:, :N].reshape(2, _KSTEPS, _NB, D2)
    g2n = g2[:N].reshape(_KSTEPS, _NB, D2)
    dinvn = dinv[:N].reshape(_KSTEPS, _NB, 1)
    W4 = fc2_W.reshape(_KSTEPS, _NB, 10, 100)
    out = _tc_post_fc2(s2n, g2n, dinvn, b2p, fc1_Wp, fc1_b, W4,
                       fc2_b.reshape(1, 100))
    return out.reshape(100)


# speedup vs baseline: 1.4794x; 1.4794x over previous
import functools

import jax
import jax.numpy as jnp
from jax import lax
from jax.experimental import pallas as pl
from jax.experimental.pallas import tpu as pltpu
from jax.experimental.pallas import tpu_sc as plsc

N = 5850
E = 93600
NP = 5888
RPT = 368
NW = 32
CH = 128
NCH = 23
EW = NCH * CH
EP = NW * EW
D1 = 8
D2 = 24

_sc_params = pltpu.CompilerParams(use_tc_tiling_on_sc=False)


@functools.cache
def _mesh():
    return plsc.VectorSubcoreMesh(
        core_axis_name="c", subcore_axis_name="s", num_cores=2, num_subcores=16
    )


def _leaky(v):
    return jnp.where(v >= 0, v, 0.01 * v)


def _sc_degree(dstp):
    @functools.partial(
        pl.kernel,
        out_type=jax.ShapeDtypeStruct((2, NP, D1), jnp.float32),
        mesh=_mesh(),
        compiler_params=_sc_params,
        scratch_types=[
            pltpu.VMEM((NCH, CH), jnp.int32),
            pltpu.VMEM((CH, D1), jnp.float32),
            pltpu.VMEM_SHARED((NP, D1), jnp.float32),
            pltpu.SemaphoreType.DMA,
        ],
    )
    def k(dstp_hbm, z_hbm, ones_hbm, out_hbm, idx_d, onesb, acc, sem):
        c = lax.axis_index("c")
        s = lax.axis_index("s")
        wid = c * 16 + s
        row = pl.ds(s * RPT, RPT)
        pltpu.sync_copy(z_hbm, acc.at[row, :])
        pltpu.sync_copy(ones_hbm, onesb)
        pltpu.sync_copy(dstp_hbm.at[wid], idx_d)
        plsc.subcore_barrier()
        for j in range(NCH):
            pltpu.sync_copy(onesb, acc.at[idx_d.at[j]], add=True)
        plsc.subcore_barrier()
        pltpu.sync_copy(acc.at[row, :], out_hbm.at[c].at[row, :])

    return k(dstp, jnp.zeros((RPT, D1), jnp.float32),
             jnp.ones((CH, D1), jnp.float32))


def _sc_scatter(g, srcp, dstp, d):
    @functools.partial(
        pl.kernel,
        out_type=jax.ShapeDtypeStruct((2, NP, d), jnp.float32),
        mesh=_mesh(),
        compiler_params=_sc_params,
        scratch_types=[
            pltpu.VMEM((NCH, CH), jnp.int32),
            pltpu.VMEM((NCH, CH), jnp.int32),
            pltpu.VMEM((EW, d), jnp.float32),
            pltpu.VMEM_SHARED((NP, d), jnp.float32),
            pltpu.VMEM_SHARED((NP, d), jnp.float32),
            pltpu.SemaphoreType.DMA,
        ],
    )
    def k(g_hbm, srcp_hbm, dstp_hbm, z_hbm, out_hbm, idx_s, idx_d, rows, g_sh, acc, sem):
        c = lax.axis_index("c")
        s = lax.axis_index("s")
        wid = c * 16 + s
        row = pl.ds(s * RPT, RPT)
        pltpu.sync_copy(z_hbm, acc.at[row, :])
        pltpu.sync_copy(g_hbm.at[row, :], g_sh.at[row, :])
        pltpu.sync_copy(srcp_hbm.at[wid], idx_s)
        pltpu.sync_copy(dstp_hbm.at[wid], idx_d)
        plsc.subcore_barrier()
        cps = [
            pltpu.async_copy(g_sh.at[idx_s.at[j]], rows.at[pl.ds(j * CH, CH), :], sem)
            for j in range(NCH)
        ]
        for cp in cps:
            cp.wait()
        for j in range(NCH):
            pltpu.sync_copy(rows.at[pl.ds(j * CH, CH), :], acc.at[idx_d.at[j]], add=True)
        plsc.subcore_barrier()
        pltpu.sync_copy(acc.at[row, :], out_hbm.at[c].at[row, :])

    return k(g, srcp, dstp, jnp.zeros((RPT, d), jnp.float32))


def _tc_h1(posp, W1p):
    def body(pos_ref, w_ref, h1_ref):
        h1_ref[...] = jnp.dot(pos_ref[...], w_ref[...],
                              preferred_element_type=jnp.float32)

    return pl.pallas_call(
        body,
        out_shape=jax.ShapeDtypeStruct((NP, D1), jnp.float32),
    )(posp, W1p)


def _tc_scale(h1, degp):
    def body(h1_ref, deg_ref, g1_ref, dinv_ref):
        deg = deg_ref[0, :, 0:1] + deg_ref[1, :, 0:1] + 1.0
        dinv = lax.rsqrt(deg)
        g1_ref[...] = dinv * h1_ref[...]
        dinv_ref[...] = dinv

    return pl.pallas_call(
        body,
        out_shape=(
            jax.ShapeDtypeStruct((NP, D1), jnp.float32),
            jax.ShapeDtypeStruct((NP, 1), jnp.float32),
        ),
    )(h1, degp)


def _tc_mid(s1p, g1, dinv, b1p, W2p):
    def body(s_ref, g1_ref, dinv_ref, b1_ref, w2_ref, g2_ref):
        dinv = dinv_ref[...]
        x1 = _leaky(dinv * (s_ref[0] + s_ref[1] + g1_ref[...]) + b1_ref[...])
        h2 = jnp.dot(x1, w2_ref[...], preferred_element_type=jnp.float32)
        g2_ref[...] = dinv * h2

    return pl.pallas_call(
        body,
        out_shape=jax.ShapeDtypeStruct((NP, D2), jnp.float32),
    )(s1p, g1, dinv, b1p, W2p)


def _tc_post(s2p, g2, dinv, b2p, fc1_Wp, fc1_b):
    def body(s_ref, g2_ref, dinv_ref, b2_ref, w_ref, b_ref, x3_ref):
        x2 = _leaky(dinv_ref[...] * (s_ref[0] + s_ref[1] + g2_ref[...]) + b2_ref[...])
        x3 = jnp.dot(x2, w_ref[...], preferred_element_type=jnp.float32)
        x3_ref[...] = _leaky(x3 + b_ref[...])

    return pl.pallas_call(
        body,
        out_shape=jax.ShapeDtypeStruct((NP, 10), jnp.float32),
    )(s2p, g2, dinv, b2p, fc1_Wp, fc1_b)


_KB = 4875
_KSTEPS = 12


def _tc_fc2(x3d, fc2_W3d, fc2_b2d):
    def body(x_ref, w_ref, b_ref, out_ref):
        kk = pl.program_id(0)
        part = jnp.dot(x_ref[0], w_ref[0], preferred_element_type=jnp.float32)
        acc = jnp.where(kk == 0, part, out_ref[...] + part)
        out_ref[...] = jnp.where(kk == _KSTEPS - 1, _leaky(acc + b_ref[...]), acc)

    return pl.pallas_call(
        body,
        grid=(_KSTEPS,),
        in_specs=[
            pl.BlockSpec((1, 1, _KB), lambda k: (k, 0, 0)),
            pl.BlockSpec((1, _KB, 100), lambda k: (k, 0, 0)),
            pl.BlockSpec((1, 100), lambda k: (0, 0)),
        ],
        out_specs=pl.BlockSpec((1, 100), lambda k: (0, 0)),
        out_shape=jax.ShapeDtypeStruct((1, 100), jnp.float32),
    )(x3d, fc2_W3d, fc2_b2d)


def kernel(pos, edge_index, W1, b1, W2, b2, fc1_W, fc1_b, fc2_W, fc2_b):
    pad = N + (jnp.arange(EP - E, dtype=jnp.int32) % 8)
    srcp = jnp.concatenate([edge_index[0], pad]).reshape(NW, NCH, CH)
    dstp = jnp.concatenate([edge_index[1], pad]).reshape(NW, NCH, CH)
    posp = jnp.pad(pos, ((0, NP - N), (0, 0)))
    W1p = jnp.pad(W1, ((0, 0), (0, D1 - 5)))
    b1p = jnp.pad(b1, (0, D1 - 5))
    W2p = jnp.pad(W2, ((0, D1 - 5), (0, D2 - 20)))
    b2p = jnp.pad(b2, (0, D2 - 20))
    fc1_Wp = jnp.pad(fc1_W, ((0, D2 - 20), (0, 0)))

    h1 = _tc_h1(posp, W1p)
    degp = _sc_degree(dstp)
    g1, dinv = _tc_scale(h1, degp)
    s1p = _sc_scatter(g1, srcp, dstp, D1)
    g2 = _tc_mid(s1p, g1, dinv, b1p, W2p)
    s2p = _sc_scatter(g2, srcp, dstp, D2)
    x3 = _tc_post(s2p, g2, dinv, b2p, fc1_Wp, fc1_b)
    x3d = x3[:N].reshape(_KSTEPS, 1, _KB)
    out = _tc_fc2(x3d, fc2_W.reshape(_KSTEPS, _KB, 100), fc2_b.reshape(1, 100))
    return out.reshape(100)


# speedup vs baseline: 1.8661x; 1.2614x over previous
import functools

import jax
import jax.numpy as jnp
from jax import lax
from jax.experimental import pallas as pl
from jax.experimental.pallas import tpu as pltpu
from jax.experimental.pallas import tpu_sc as plsc

N = 5850
E = 93600
NP = 5888
RPT = 368
NW = 32
CH = 128
NCH = 23
EW = NCH * CH
EP = NW * EW
D1 = 8
D2 = 24

_sc_params = pltpu.CompilerParams(use_tc_tiling_on_sc=False)


@functools.cache
def _mesh():
    return plsc.VectorSubcoreMesh(
        core_axis_name="c", subcore_axis_name="s", num_cores=2, num_subcores=16
    )


def _leaky(v):
    return jnp.where(v >= 0, v, 0.01 * v)


def _sc_degree(dstp):
    @functools.partial(
        pl.kernel,
        out_type=jax.ShapeDtypeStruct((2, NP, D1), jnp.float32),
        mesh=_mesh(),
        compiler_params=_sc_params,
        scratch_types=[
            pltpu.VMEM((NCH, CH), jnp.int32),
            pltpu.VMEM((CH, D1), jnp.float32),
            pltpu.VMEM_SHARED((NP, D1), jnp.float32),
            pltpu.SemaphoreType.DMA,
        ],
    )
    def k(dstp_hbm, z_hbm, ones_hbm, out_hbm, idx_d, onesb, acc, sem):
        c = lax.axis_index("c")
        s = lax.axis_index("s")
        wid = c * 16 + s
        row = pl.ds(s * RPT, RPT)
        pltpu.sync_copy(z_hbm, acc.at[row, :])
        pltpu.sync_copy(ones_hbm, onesb)
        pltpu.sync_copy(dstp_hbm.at[wid], idx_d)
        plsc.subcore_barrier()
        for j in range(NCH):
            pltpu.sync_copy(onesb, acc.at[idx_d.at[j]], add=True)
        plsc.subcore_barrier()
        pltpu.sync_copy(acc.at[row, :], out_hbm.at[c].at[row, :])

    return k(dstp, jnp.zeros((RPT, D1), jnp.float32),
             jnp.ones((CH, D1), jnp.float32))


def _sc_scatter(g, srcp, dstp, d):
    @functools.partial(
        pl.kernel,
        out_type=jax.ShapeDtypeStruct((2, NP, d), jnp.float32),
        mesh=_mesh(),
        compiler_params=_sc_params,
        scratch_types=[
            pltpu.VMEM((NCH, CH), jnp.int32),
            pltpu.VMEM((NCH, CH), jnp.int32),
            pltpu.VMEM((EW, d), jnp.float32),
            pltpu.VMEM_SHARED((NP, d), jnp.float32),
            pltpu.VMEM_SHARED((NP, d), jnp.float32),
            pltpu.SemaphoreType.DMA,
        ],
    )
    def k(g_hbm, srcp_hbm, dstp_hbm, z_hbm, out_hbm, idx_s, idx_d, rows, g_sh, acc, sem):
        c = lax.axis_index("c")
        s = lax.axis_index("s")
        wid = c * 16 + s
        row = pl.ds(s * RPT, RPT)
        pltpu.sync_copy(z_hbm, acc.at[row, :])
        pltpu.sync_copy(g_hbm.at[row, :], g_sh.at[row, :])
        pltpu.sync_copy(srcp_hbm.at[wid], idx_s)
        pltpu.sync_copy(dstp_hbm.at[wid], idx_d)
        plsc.subcore_barrier()
        cps = [
            pltpu.async_copy(g_sh.at[idx_s.at[j]], rows.at[pl.ds(j * CH, CH), :], sem)
            for j in range(NCH)
        ]
        for cp in cps:
            cp.wait()
        for j in range(NCH):
            pltpu.sync_copy(rows.at[pl.ds(j * CH, CH), :], acc.at[idx_d.at[j]], add=True)
        plsc.subcore_barrier()
        pltpu.sync_copy(acc.at[row, :], out_hbm.at[c].at[row, :])

    return k(g, srcp, dstp, jnp.zeros((RPT, d), jnp.float32))


def _tc_h1(posp, W1p):
    def body(pos_ref, w_ref, h1_ref):
        h1_ref[...] = jnp.dot(pos_ref[...], w_ref[...],
                              preferred_element_type=jnp.float32)

    return pl.pallas_call(
        body,
        out_shape=jax.ShapeDtypeStruct((NP, D1), jnp.float32),
    )(posp, W1p)


def _tc_scale(h1, degp):
    def body(h1_ref, deg_ref, g1_ref, dinv_ref):
        deg = deg_ref[0, :, 0:1] + deg_ref[1, :, 0:1] + 1.0
        dinv = lax.rsqrt(deg)
        g1_ref[...] = dinv * h1_ref[...]
        dinv_ref[...] = dinv

    return pl.pallas_call(
        body,
        out_shape=(
            jax.ShapeDtypeStruct((NP, D1), jnp.float32),
            jax.ShapeDtypeStruct((NP, 1), jnp.float32),
        ),
    )(h1, degp)


def _tc_mid(s1p, g1, dinv, b1p, W2p):
    def body(s_ref, g1_ref, dinv_ref, b1_ref, w2_ref, g2_ref):
        dinv = dinv_ref[...]
        x1 = _leaky(dinv * (s_ref[0] + s_ref[1] + g1_ref[...]) + b1_ref[...])
        h2 = jnp.dot(x1, w2_ref[...], preferred_element_type=jnp.float32)
        g2_ref[...] = dinv * h2

    return pl.pallas_call(
        body,
        out_shape=jax.ShapeDtypeStruct((NP, D2), jnp.float32),
    )(s1p, g1, dinv, b1p, W2p)


def _tc_post(s2p, g2, dinv, b2p, fc1_Wp, fc1_b):
    def body(s_ref, g2_ref, dinv_ref, b2_ref, w_ref, b_ref, x3_ref):
        x2 = _leaky(dinv_ref[...] * (s_ref[0] + s_ref[1] + g2_ref[...]) + b2_ref[...])
        x3 = jnp.dot(x2, w_ref[...], preferred_element_type=jnp.float32)
        x3_ref[...] = _leaky(x3 + b_ref[...])

    return pl.pallas_call(
        body,
        out_shape=jax.ShapeDtypeStruct((NP, 10), jnp.float32),
    )(s2p, g2, dinv, b2p, fc1_Wp, fc1_b)


def _tc_fc2(x2d, fc2_W, fc2_b2d):
    def body(x_ref, w_ref, b_ref, out_ref):
        part = jnp.dot(x_ref[...], w_ref[...], preferred_element_type=jnp.float32)
        out_ref[...] = _leaky(part + b_ref[...])

    return pl.pallas_call(
        body,
        out_shape=jax.ShapeDtypeStruct((1, 100), jnp.float32),
    )(x2d, fc2_W, fc2_b2d)


def kernel(pos, edge_index, W1, b1, W2, b2, fc1_W, fc1_b, fc2_W, fc2_b):
    pad = N + (jnp.arange(EP - E, dtype=jnp.int32) % 8)
    srcp = jnp.concatenate([edge_index[0], pad]).reshape(NW, NCH, CH)
    dstp = jnp.concatenate([edge_index[1], pad]).reshape(NW, NCH, CH)
    posp = jnp.pad(pos, ((0, NP - N), (0, 0)))
    W1p = jnp.pad(W1, ((0, 0), (0, D1 - 5)))
    b1p = jnp.pad(b1, (0, D1 - 5))
    W2p = jnp.pad(W2, ((0, D1 - 5), (0, D2 - 20)))
    b2p = jnp.pad(b2, (0, D2 - 20))
    fc1_Wp = jnp.pad(fc1_W, ((0, D2 - 20), (0, 0)))

    h1 = _tc_h1(posp, W1p)
    degp = _sc_degree(dstp)
    g1, dinv = _tc_scale(h1, degp)
    s1p = _sc_scatter(g1, srcp, dstp, D1)
    g2 = _tc_mid(s1p, g1, dinv, b1p, W2p)
    s2p = _sc_scatter(g2, srcp, dstp, D2)
    x3 = _tc_post(s2p, g2, dinv, b2p, fc1_Wp, fc1_b)
    x2d = x3[:N].reshape(1, N * 10)
    out = _tc_fc2(x2d, fc2_W, fc2_b.reshape(1, 100))
    return out.reshape(100)


# speedup vs baseline: 1.8926x; 1.0142x over previous
import functools

import jax
import jax.numpy as jnp
from jax import lax
from jax.experimental import pallas as pl
from jax.experimental.pallas import tpu as pltpu
from jax.experimental.pallas import tpu_sc as plsc

N = 5850
E = 93600
NP = 5888
RPT = 368
NW = 32
CH = 128
NCH = 23
EW = NCH * CH
EP = NW * EW
D1 = 8
D2 = 24

_sc_params = pltpu.CompilerParams(use_tc_tiling_on_sc=False)


@functools.cache
def _mesh():
    return plsc.VectorSubcoreMesh(
        core_axis_name="c", subcore_axis_name="s", num_cores=2, num_subcores=16
    )


def _leaky(v):
    return jnp.where(v >= 0, v, 0.01 * v)


def _sc_degree(dstp):
    @functools.partial(
        pl.kernel,
        out_type=jax.ShapeDtypeStruct((2, NP, D1), jnp.float32),
        mesh=_mesh(),
        compiler_params=_sc_params,
        scratch_types=[
            pltpu.VMEM((NCH, CH), jnp.int32),
            pltpu.VMEM((CH, D1), jnp.float32),
            pltpu.VMEM_SHARED((NP, D1), jnp.float32),
            pltpu.SemaphoreType.DMA,
        ],
    )
    def k(dstp_hbm, z_hbm, ones_hbm, out_hbm, idx_d, onesb, acc, sem):
        c = lax.axis_index("c")
        s = lax.axis_index("s")
        wid = c * 16 + s
        row = pl.ds(s * RPT, RPT)
        pltpu.sync_copy(z_hbm, acc.at[row, :])
        pltpu.sync_copy(ones_hbm, onesb)
        pltpu.sync_copy(dstp_hbm.at[wid], idx_d)
        plsc.subcore_barrier()
        adds = [
            pltpu.async_copy(onesb, acc.at[idx_d.at[j]], sem, add=True)
            for j in range(NCH)
        ]
        for cp in adds:
            cp.wait()
        plsc.subcore_barrier()
        pltpu.sync_copy(acc.at[row, :], out_hbm.at[c].at[row, :])

    return k(dstp, jnp.zeros((RPT, D1), jnp.float32),
             jnp.ones((CH, D1), jnp.float32))


def _sc_scatter(g, srcp, dstp, d):
    @functools.partial(
        pl.kernel,
        out_type=jax.ShapeDtypeStruct((2, NP, d), jnp.float32),
        mesh=_mesh(),
        compiler_params=_sc_params,
        scratch_types=[
            pltpu.VMEM((NCH, CH), jnp.int32),
            pltpu.VMEM((NCH, CH), jnp.int32),
            pltpu.VMEM((EW, d), jnp.float32),
            pltpu.VMEM_SHARED((NP, d), jnp.float32),
            pltpu.VMEM_SHARED((NP, d), jnp.float32),
            pltpu.SemaphoreType.DMA,
            pltpu.SemaphoreType.DMA,
        ],
    )
    def k(g_hbm, srcp_hbm, dstp_hbm, z_hbm, out_hbm, idx_s, idx_d, rows, g_sh, acc,
          sem, sem2):
        c = lax.axis_index("c")
        s = lax.axis_index("s")
        wid = c * 16 + s
        row = pl.ds(s * RPT, RPT)
        pltpu.sync_copy(z_hbm, acc.at[row, :])
        pltpu.sync_copy(g_hbm.at[row, :], g_sh.at[row, :])
        pltpu.sync_copy(srcp_hbm.at[wid], idx_s)
        pltpu.sync_copy(dstp_hbm.at[wid], idx_d)
        plsc.subcore_barrier()
        cps = [
            pltpu.async_copy(g_sh.at[idx_s.at[j]], rows.at[pl.ds(j * CH, CH), :], sem)
            for j in range(NCH)
        ]
        for cp in cps:
            cp.wait()
        adds = [
            pltpu.async_copy(rows.at[pl.ds(j * CH, CH), :], acc.at[idx_d.at[j]],
                             sem2, add=True)
            for j in range(NCH)
        ]
        for cp in adds:
            cp.wait()
        plsc.subcore_barrier()
        pltpu.sync_copy(acc.at[row, :], out_hbm.at[c].at[row, :])

    return k(g, srcp, dstp, jnp.zeros((RPT, d), jnp.float32))


def _tc_h1(posp, W1p):
    def body(pos_ref, w_ref, h1_ref):
        h1_ref[...] = jnp.dot(pos_ref[...], w_ref[...],
                              preferred_element_type=jnp.float32)

    return pl.pallas_call(
        body,
        out_shape=jax.ShapeDtypeStruct((NP, D1), jnp.float32),
    )(posp, W1p)


def _tc_scale(h1, degp):
    def body(h1_ref, deg_ref, g1_ref, dinv_ref):
        deg = deg_ref[0, :, 0:1] + deg_ref[1, :, 0:1] + 1.0
        dinv = lax.rsqrt(deg)
        g1_ref[...] = dinv * h1_ref[...]
        dinv_ref[...] = dinv

    return pl.pallas_call(
        body,
        out_shape=(
            jax.ShapeDtypeStruct((NP, D1), jnp.float32),
            jax.ShapeDtypeStruct((NP, 1), jnp.float32),
        ),
    )(h1, degp)


def _tc_mid(s1p, g1, dinv, b1p, W2p):
    def body(s_ref, g1_ref, dinv_ref, b1_ref, w2_ref, g2_ref):
        dinv = dinv_ref[...]
        x1 = _leaky(dinv * (s_ref[0] + s_ref[1] + g1_ref[...]) + b1_ref[...])
        h2 = jnp.dot(x1, w2_ref[...], preferred_element_type=jnp.float32)
        g2_ref[...] = dinv * h2

    return pl.pallas_call(
        body,
        out_shape=jax.ShapeDtypeStruct((NP, D2), jnp.float32),
    )(s1p, g1, dinv, b1p, W2p)


def _tc_post(s2p, g2, dinv, b2p, fc1_Wp, fc1_b):
    def body(s_ref, g2_ref, dinv_ref, b2_ref, w_ref, b_ref, x3_ref):
        x2 = _leaky(dinv_ref[...] * (s_ref[0] + s_ref[1] + g2_ref[...]) + b2_ref[...])
        x3 = jnp.dot(x2, w_ref[...], preferred_element_type=jnp.float32)
        x3_ref[...] = _leaky(x3 + b_ref[...])

    return pl.pallas_call(
        body,
        out_shape=jax.ShapeDtypeStruct((NP, 10), jnp.float32),
    )(s2p, g2, dinv, b2p, fc1_Wp, fc1_b)


def _tc_fc2(x2d, fc2_W, fc2_b2d):
    def body(x_ref, w_ref, b_ref, out_ref):
        part = jnp.dot(x_ref[...], w_ref[...], preferred_element_type=jnp.float32)
        out_ref[...] = _leaky(part + b_ref[...])

    return pl.pallas_call(
        body,
        out_shape=jax.ShapeDtypeStruct((1, 100), jnp.float32),
    )(x2d, fc2_W, fc2_b2d)


def kernel(pos, edge_index, W1, b1, W2, b2, fc1_W, fc1_b, fc2_W, fc2_b):
    pad = N + (jnp.arange(EP - E, dtype=jnp.int32) % 8)
    srcp = jnp.concatenate([edge_index[0], pad]).reshape(NW, NCH, CH)
    dstp = jnp.concatenate([edge_index[1], pad]).reshape(NW, NCH, CH)
    posp = jnp.pad(pos, ((0, NP - N), (0, 0)))
    W1p = jnp.pad(W1, ((0, 0), (0, D1 - 5)))
    b1p = jnp.pad(b1, (0, D1 - 5))
    W2p = jnp.pad(W2, ((0, D1 - 5), (0, D2 - 20)))
    b2p = jnp.pad(b2, (0, D2 - 20))
    fc1_Wp = jnp.pad(fc1_W, ((0, D2 - 20), (0, 0)))

    h1 = _tc_h1(posp, W1p)
    degp = _sc_degree(dstp)
    g1, dinv = _tc_scale(h1, degp)
    s1p = _sc_scatter(g1, srcp, dstp, D1)
    g2 = _tc_mid(s1p, g1, dinv, b1p, W2p)
    s2p = _sc_scatter(g2, srcp, dstp, D2)
    x3 = _tc_post(s2p, g2, dinv, b2p, fc1_Wp, fc1_b)
    x2d = x3[:N].reshape(1, N * 10)
    out = _tc_fc2(x2d, fc2_W, fc2_b.reshape(1, 100))
    return out.reshape(100)


# speedup vs baseline: 1.9276x; 1.0185x over previous
import functools

import jax
import jax.numpy as jnp
from jax import lax
from jax.experimental import pallas as pl
from jax.experimental.pallas import tpu as pltpu
from jax.experimental.pallas import tpu_sc as plsc

N = 5850
E = 93600
NP = 5888
RPT = 368
NW = 32
CH = 128
NCH = 23
EW = NCH * CH
EP = NW * EW
D1 = 8
D2 = 24

_sc_params = pltpu.CompilerParams(use_tc_tiling_on_sc=False)


@functools.cache
def _mesh():
    return plsc.VectorSubcoreMesh(
        core_axis_name="c", subcore_axis_name="s", num_cores=2, num_subcores=16
    )


def _leaky(v):
    return jnp.where(v >= 0, v, 0.01 * v)


def _sc_degree(dstp):
    @functools.partial(
        pl.kernel,
        out_type=jax.ShapeDtypeStruct((2, NP, D1), jnp.float32),
        mesh=_mesh(),
        compiler_params=_sc_params,
        scratch_types=[
            pltpu.VMEM((NCH, CH), jnp.int32),
            pltpu.VMEM((CH, D1), jnp.float32),
            pltpu.VMEM_SHARED((NP, D1), jnp.float32),
            pltpu.SemaphoreType.DMA,
        ],
    )
    def k(dstp_hbm, z_hbm, ones_hbm, out_hbm, idx_d, onesb, acc, sem):
        c = lax.axis_index("c")
        s = lax.axis_index("s")
        wid = c * 16 + s
        row = pl.ds(s * RPT, RPT)
        pltpu.sync_copy(z_hbm, acc.at[row, :])
        pltpu.sync_copy(ones_hbm, onesb)
        pltpu.sync_copy(dstp_hbm.at[wid], idx_d)
        plsc.subcore_barrier()
        adds = [
            pltpu.async_copy(onesb, acc.at[idx_d.at[j]], sem, add=True)
            for j in range(NCH)
        ]
        for cp in adds:
            cp.wait()
        plsc.subcore_barrier()
        pltpu.sync_copy(acc.at[row, :], out_hbm.at[c].at[row, :])

    return k(dstp, jnp.zeros((RPT, D1), jnp.float32),
             jnp.ones((CH, D1), jnp.float32))


def _sc_scatter(g, srcp, dstp, d):
    @functools.partial(
        pl.kernel,
        out_type=jax.ShapeDtypeStruct((2, NP, d), jnp.float32),
        mesh=_mesh(),
        compiler_params=_sc_params,
        scratch_types=[
            pltpu.VMEM((NCH, CH), jnp.int32),
            pltpu.VMEM((NCH, CH), jnp.int32),
            pltpu.VMEM((EW, d), jnp.float32),
            pltpu.VMEM_SHARED((NP, d), jnp.float32),
            pltpu.VMEM_SHARED((NP, d), jnp.float32),
            pltpu.SemaphoreType.DMA,
            pltpu.SemaphoreType.DMA,
        ],
    )
    def k(g_hbm, srcp_hbm, dstp_hbm, z_hbm, out_hbm, idx_s, idx_d, rows, g_sh, acc,
          sem, sem2):
        c = lax.axis_index("c")
        s = lax.axis_index("s")
        wid = c * 16 + s
        row = pl.ds(s * RPT, RPT)
        pltpu.sync_copy(z_hbm, acc.at[row, :])
        pltpu.sync_copy(g_hbm.at[row, :], g_sh.at[row, :])
        pltpu.sync_copy(srcp_hbm.at[wid], idx_s)
        pltpu.sync_copy(dstp_hbm.at[wid], idx_d)
        plsc.subcore_barrier()
        cps = [
            pltpu.async_copy(g_sh.at[idx_s.at[j]], rows.at[pl.ds(j * CH, CH), :], sem)
            for j in range(NCH)
        ]
        for cp in cps:
            cp.wait()
        adds = [
            pltpu.async_copy(rows.at[pl.ds(j * CH, CH), :], acc.at[idx_d.at[j]],
                             sem2, add=True)
            for j in range(NCH)
        ]
        for cp in adds:
            cp.wait()
        plsc.subcore_barrier()
        pltpu.sync_copy(acc.at[row, :], out_hbm.at[c].at[row, :])

    return k(g, srcp, dstp, jnp.zeros((RPT, d), jnp.float32))


def _tc_pre(posp, degp, W1):
    def body(pos_ref, deg_ref, w_ref, g1_ref, dinv_ref):
        deg = deg_ref[0, :, 0:1] + deg_ref[1, :, 0:1] + 1.0
        dinv = lax.rsqrt(deg)
        w = jnp.pad(w_ref[...], ((0, 0), (0, D1 - 5)))
        h1 = jnp.dot(pos_ref[...], w, preferred_element_type=jnp.float32)
        g1_ref[...] = dinv * h1
        dinv_ref[...] = dinv

    return pl.pallas_call(
        body,
        out_shape=(
            jax.ShapeDtypeStruct((NP, D1), jnp.float32),
            jax.ShapeDtypeStruct((NP, 1), jnp.float32),
        ),
    )(posp, degp, W1)


def _tc_mid(s1p, g1, dinv, b1, W2):
    def body(s_ref, g1_ref, dinv_ref, b1_ref, w2_ref, g2_ref):
        dinv = dinv_ref[...]
        b1p = jnp.pad(b1_ref[...], (0, D1 - 5))
        w2p = jnp.pad(w2_ref[...], ((0, D1 - 5), (0, D2 - 20)))
        x1 = _leaky(dinv * (s_ref[0] + s_ref[1] + g1_ref[...]) + b1p)
        h2 = jnp.dot(x1, w2p, preferred_element_type=jnp.float32)
        g2_ref[...] = dinv * h2

    return pl.pallas_call(
        body,
        out_shape=jax.ShapeDtypeStruct((NP, D2), jnp.float32),
    )(s1p, g1, dinv, b1, W2)


def _tc_post(s2p, g2, dinv, b2, fc1_W, fc1_b):
    def body(s_ref, g2_ref, dinv_ref, b2_ref, w_ref, b_ref, x3_ref):
        b2p = jnp.pad(b2_ref[...], (0, D2 - 20))
        wp = jnp.pad(w_ref[...], ((0, D2 - 20), (0, 0)))
        x2 = _leaky(dinv_ref[...] * (s_ref[0] + s_ref[1] + g2_ref[...]) + b2p)
        x3 = jnp.dot(x2, wp, preferred_element_type=jnp.float32)
        x3_ref[...] = _leaky(x3 + b_ref[...])

    return pl.pallas_call(
        body,
        out_shape=jax.ShapeDtypeStruct((NP, 10), jnp.float32),
    )(s2p, g2, dinv, b2, fc1_W, fc1_b)


def _tc_fc2(x2d, fc2_W, fc2_b2d):
    def body(x_ref, w_ref, b_ref, out_ref):
        part = jnp.dot(x_ref[...], w_ref[...], preferred_element_type=jnp.float32)
        out_ref[...] = _leaky(part + b_ref[...])

    return pl.pallas_call(
        body,
        out_shape=jax.ShapeDtypeStruct((1, 100), jnp.float32),
    )(x2d, fc2_W, fc2_b2d)


def kernel(pos, edge_index, W1, b1, W2, b2, fc1_W, fc1_b, fc2_W, fc2_b):
    pad = N + (jnp.arange(EP - E, dtype=jnp.int32) % 8)
    srcp = jnp.concatenate([edge_index[0], pad]).reshape(NW, NCH, CH)
    dstp = jnp.concatenate([edge_index[1], pad]).reshape(NW, NCH, CH)
    posp = jnp.pad(pos, ((0, NP - N), (0, 0)))

    degp = _sc_degree(dstp)
    g1, dinv = _tc_pre(posp, degp, W1)
    s1p = _sc_scatter(g1, srcp, dstp, D1)
    g2 = _tc_mid(s1p, g1, dinv, b1, W2)
    s2p = _sc_scatter(g2, srcp, dstp, D2)
    x3 = _tc_post(s2p, g2, dinv, b2, fc1_W, fc1_b)
    x2d = x3[:N].reshape(1, N * 10)
    out = _tc_fc2(x2d, fc2_W, fc2_b.reshape(1, 100))
    return out.reshape(100)
